# baseline (speedup 1.0000x reference)
import functools

import jax
import jax.numpy as jnp
from jax import lax
from jax.experimental import pallas as pl
from jax.experimental.pallas import tpu as pltpu
from jax.experimental.pallas import tpu_sc as plsc

L = 16
CHUNK = 2048
GROUP = 128


@functools.lru_cache(maxsize=None)
def _build(n, h, w):
    info = plsc.get_sparse_core_info()
    nc, ns = info.num_cores, info.num_subcores
    nw = nc * ns
    assert n % (nw * 2 * CHUNK) == 0
    nq = n // nw
    nchunk = nq // CHUNK
    npair = nchunk // 2
    groups = CHUNK // GROUP
    gvecs = GROUP // L
    seg = (h * w) // ns
    wf = float(w)
    hf = float(h)

    mesh = plsc.VectorSubcoreMesh(core_axis_name="c", subcore_axis_name="s")

    @functools.partial(
        pl.kernel,
        out_type=jax.ShapeDtypeStruct((n,), jnp.float32),
        mesh=mesh,
        scratch_types=[
            pltpu.VMEM_SHARED((h * w,), jnp.float32),
            [pltpu.VMEM((CHUNK,), jnp.float32)] * 2,
            [pltpu.VMEM((CHUNK,), jnp.float32)] * 2,
            pltpu.VMEM((CHUNK,), jnp.int32),
            pltpu.VMEM((CHUNK,), jnp.int32),
            pltpu.VMEM((CHUNK,), jnp.int32),
            pltpu.VMEM((CHUNK,), jnp.int32),
            pltpu.VMEM((CHUNK,), jnp.float32),
            pltpu.VMEM((CHUNK,), jnp.float32),
            pltpu.VMEM((CHUNK,), jnp.float32),
            pltpu.VMEM((CHUNK,), jnp.float32),
            pltpu.VMEM((CHUNK,), jnp.float32),
            pltpu.VMEM((CHUNK,), jnp.float32),
            pltpu.VMEM((CHUNK,), jnp.float32),
            pltpu.VMEM((CHUNK,), jnp.float32),
            pltpu.VMEM((5 * L,), jnp.float32),
            [pltpu.SemaphoreType.DMA] * (CHUNK // GROUP),
            [pltpu.SemaphoreType.DMA] * 2,
        ],
    )
    def run(xf, yf, img, params, out, shared, xvs, yvs, idxa, idxc, idxb,
            idxd, wdx0, wdy0, wdy1, fav, fcv, fbv, fdv, outv, pv, gsems,
            xysems):
        cid = lax.axis_index("c")
        sid = lax.axis_index("s")
        wid = sid * nc + cid
        sbase = sid * seg
        pltpu.sync_copy(img.at[pl.ds(sbase, seg)], shared.at[pl.ds(sbase, seg)])
        pltpu.sync_copy(params, pv)
        plsc.subcore_barrier()
        axv = pv[pl.ds(0 * L, L)]
        bxv = pv[pl.ds(1 * L, L)]
        ayv = pv[pl.ds(2 * L, L)]
        byv = pv[pl.ds(3 * L, L)]
        sclv = pv[pl.ds(4 * L, L)]
        base0 = wid * nq

        def fire_xy(t, s):
            b = base0 + t * CHUNK
            pltpu.async_copy(xf.at[pl.ds(b, CHUNK)], xvs[s], xysems[s])
            pltpu.async_copy(yf.at[pl.ds(b, CHUNK)], yvs[s], xysems[s])

        def wait_xy(t, s):
            b = base0 + t * CHUNK
            for src, dst in ((xf, xvs[s]), (yf, yvs[s])):
                pltpu.make_async_copy(
                    src.at[pl.ds(b, CHUNK)], dst, xysems[s]).wait()

        def process(t, s):
            base = base0 + t * CHUNK
            xv = xvs[s]
            yv = yvs[s]
            fired = []
            for g in range(groups):
                goff = g * GROUP

                @pl.loop(0, gvecs, unroll=2)
                def _pass_a(i):
                    off = goff + i * L
                    sl = pl.ds(off, L)
                    xq = xv[sl]
                    yq = yv[sl]
                    xp = xq * axv + bxv
                    yp = yq * ayv + byv
                    oob = ((yp < -0.5) | (yp > hf - 0.5)
                           | (xp < -0.5) | (xp > wf - 0.5))
                    xpc = jnp.minimum(jnp.maximum(xp, 0.0), wf - 2.0)
                    ypc = jnp.minimum(jnp.maximum(yp, 0.0), hf - 2.0)
                    x0i = xpc.astype(jnp.int32)
                    y0i = ypc.astype(jnp.int32)
                    dx0 = xp - x0i.astype(jnp.float32)
                    dy0 = yp - y0i.astype(jnp.float32)
                    dy1 = 1.0 - dy0
                    zero = jnp.zeros((L,), jnp.float32)
                    ia = y0i * w + x0i
                    idxa[sl] = ia
                    idxc[sl] = ia + 1
                    idxb[sl] = ia + w
                    idxd[sl] = ia + (w + 1)
                    wdx0[sl] = dx0
                    wdy0[sl] = jnp.where(oob, zero, dy0 * sclv)
                    wdy1[sl] = jnp.where(oob, zero, dy1 * sclv)

                gsl = pl.ds(goff, GROUP)
                fired.append([
                    pltpu.async_copy(shared.at[ix.at[gsl]], buf.at[gsl],
                                     gsems[g])
                    for ix, buf in ((idxa, fav), (idxc, fcv),
                                    (idxb, fbv), (idxd, fdv))
                ])

            for g in range(groups):
                goff = g * GROUP
                for cp in fired[g]:
                    cp.wait()

                @pl.loop(0, gvecs, unroll=2)
                def _pass_b(i):
                    off = goff + i * L
                    sl = pl.ds(off, L)
                    dx0 = wdx0[sl]
                    dy0 = wdy0[sl]
                    dy1 = wdy1[sl]
                    dx1 = 1.0 - dx0
                    t1 = fav[sl] * dx1 + fcv[sl] * dx0
                    t0 = fbv[sl] * dx1 + fdv[sl] * dx0
                    outv[sl] = dy1 * t1 + dy0 * t0

            pltpu.sync_copy(outv, out.at[pl.ds(base, CHUNK)])

        fire_xy(0, 0)

        @pl.loop(0, npair)
        def _pair(u):
            t0 = u * 2
            t1 = t0 + 1
            wait_xy(t0, 0)
            fire_xy(t1, 1)
            process(t0, 0)
            wait_xy(t1, 1)

            @pl.when(u + 1 < npair)
            def _pf():
                fire_xy(t1 + 1, 0)

            process(t1, 1)

    return run


def kernel(x, y, x0, y0, image, pixelscale, scale):
    h, w = image.shape
    n = x.size
    xf = x.reshape(-1)
    yf = y.reshape(-1)
    img = image.reshape(-1)
    fov_x = pixelscale * w
    fov_y = pixelscale * h
    ax = jnp.float32(w) / fov_x
    ay = jnp.float32(h) / fov_y
    bx = jnp.float32((w - 1) * 0.5) - x0 * ax
    by = jnp.float32((h - 1) * 0.5) - y0 * ay
    params = jnp.concatenate(
        [jnp.full((L,), v, jnp.float32) for v in (ax, bx, ay, by, scale)])
    out = _build(n, h, w)(xf, yf, img, params)
    return out.reshape(x.shape)


# speedup vs baseline: 1.0216x; 1.0216x over previous
import functools

import jax
import jax.numpy as jnp
from jax import lax
from jax.experimental import pallas as pl
from jax.experimental.pallas import tpu as pltpu
from jax.experimental.pallas import tpu_sc as plsc

L = 16
CHUNK = 1024
GROUP = 128


@functools.lru_cache(maxsize=None)
def _build(n, h, w):
    info = plsc.get_sparse_core_info()
    nc, ns = info.num_cores, info.num_subcores
    nw = nc * ns
    assert n % (nw * 2 * CHUNK) == 0
    nq = n // nw
    nchunk = nq // CHUNK
    npair = nchunk // 2
    groups = CHUNK // GROUP
    gvecs = GROUP // L
    seg = (h * w) // ns
    wf = float(w)
    hf = float(h)

    mesh = plsc.VectorSubcoreMesh(core_axis_name="c", subcore_axis_name="s")

    @functools.partial(
        pl.kernel,
        out_type=jax.ShapeDtypeStruct((n,), jnp.float32),
        mesh=mesh,
        scratch_types=[
            pltpu.VMEM_SHARED((h * w,), jnp.float32),
            [pltpu.VMEM((CHUNK,), jnp.float32)] * 2,
            [pltpu.VMEM((CHUNK,), jnp.float32)] * 2,
            pltpu.VMEM((CHUNK,), jnp.int32),
            pltpu.VMEM((CHUNK,), jnp.int32),
            pltpu.VMEM((CHUNK,), jnp.int32),
            pltpu.VMEM((CHUNK,), jnp.int32),
            pltpu.VMEM((CHUNK,), jnp.float32),
            pltpu.VMEM((CHUNK,), jnp.float32),
            pltpu.VMEM((CHUNK,), jnp.float32),
            pltpu.VMEM((CHUNK,), jnp.float32),
            pltpu.VMEM((CHUNK,), jnp.float32),
            pltpu.VMEM((CHUNK,), jnp.float32),
            pltpu.VMEM((CHUNK,), jnp.float32),
            pltpu.VMEM((CHUNK,), jnp.float32),
            pltpu.VMEM((5 * L,), jnp.float32),
            [pltpu.SemaphoreType.DMA] * 8,
            [pltpu.SemaphoreType.DMA] * 2,
        ],
    )
    def run(xf, yf, img, params, out, shared, xvs, yvs, idxa, idxc, idxb,
            idxd, wdx0, wdy0, wdy1, fav, fcv, fbv, fdv, outv, pv, gsems,
            xysems):
        cid = lax.axis_index("c")
        sid = lax.axis_index("s")
        wid = sid * nc + cid
        sbase = sid * seg
        pltpu.sync_copy(img.at[pl.ds(sbase, seg)], shared.at[pl.ds(sbase, seg)])
        pltpu.sync_copy(params, pv)
        plsc.subcore_barrier()
        axv = pv[pl.ds(0 * L, L)]
        bxv = pv[pl.ds(1 * L, L)]
        ayv = pv[pl.ds(2 * L, L)]
        byv = pv[pl.ds(3 * L, L)]
        sclv = pv[pl.ds(4 * L, L)]
        base0 = wid * nq

        def fire_xy(t, s):
            b = base0 + t * CHUNK
            pltpu.async_copy(xf.at[pl.ds(b, CHUNK)], xvs[s], xysems[s])
            pltpu.async_copy(yf.at[pl.ds(b, CHUNK)], yvs[s], xysems[s])

        def wait_xy(t, s):
            b = base0 + t * CHUNK
            for src, dst in ((xf, xvs[s]), (yf, yvs[s])):
                pltpu.make_async_copy(
                    src.at[pl.ds(b, CHUNK)], dst, xysems[s]).wait()

        def process(t, s):
            base = base0 + t * CHUNK
            xv = xvs[s]
            yv = yvs[s]
            fired = []
            for g in range(groups):
                goff = g * GROUP

                @plsc.parallel_loop(0, gvecs, unroll=2)
                def _pass_a(i):
                    off = goff + i * L
                    sl = pl.ds(off, L)
                    xq = xv[sl]
                    yq = yv[sl]
                    xp = xq * axv + bxv
                    yp = yq * ayv + byv
                    oob = ((yp < -0.5) | (yp > hf - 0.5)
                           | (xp < -0.5) | (xp > wf - 0.5))
                    xpc = jnp.minimum(jnp.maximum(xp, 0.0), wf - 2.0)
                    ypc = jnp.minimum(jnp.maximum(yp, 0.0), hf - 2.0)
                    x0i = xpc.astype(jnp.int32)
                    y0i = ypc.astype(jnp.int32)
                    dx0 = xp - x0i.astype(jnp.float32)
                    dy0 = yp - y0i.astype(jnp.float32)
                    dy1 = 1.0 - dy0
                    zero = jnp.zeros((L,), jnp.float32)
                    ia = y0i * w + x0i
                    idxa[sl] = ia
                    idxc[sl] = ia + 1
                    idxb[sl] = ia + w
                    idxd[sl] = ia + (w + 1)
                    wdx0[sl] = dx0
                    wdy0[sl] = jnp.where(oob, zero, dy0 * sclv)
                    wdy1[sl] = jnp.where(oob, zero, dy1 * sclv)

                gsl = pl.ds(goff, GROUP)
                fired.append([
                    pltpu.async_copy(shared.at[ix.at[gsl]], buf.at[gsl],
                                     gsems[g])
                    for ix, buf in ((idxa, fav), (idxc, fcv),
                                    (idxb, fbv), (idxd, fdv))
                ])

            for g in range(groups):
                goff = g * GROUP
                for cp in fired[g]:
                    cp.wait()

                @plsc.parallel_loop(0, gvecs, unroll=2)
                def _pass_b(i):
                    off = goff + i * L
                    sl = pl.ds(off, L)
                    dx0 = wdx0[sl]
                    dy0 = wdy0[sl]
                    dy1 = wdy1[sl]
                    dx1 = 1.0 - dx0
                    t1 = fav[sl] * dx1 + fcv[sl] * dx0
                    t0 = fbv[sl] * dx1 + fdv[sl] * dx0
                    outv[sl] = dy1 * t1 + dy0 * t0

            pltpu.sync_copy(outv, out.at[pl.ds(base, CHUNK)])

        fire_xy(0, 0)

        @pl.loop(0, npair)
        def _pair(u):
            t0 = u * 2
            t1 = t0 + 1
            wait_xy(t0, 0)
            fire_xy(t1, 1)
            process(t0, 0)
            wait_xy(t1, 1)

            @pl.when(u + 1 < npair)
            def _pf():
                fire_xy(t1 + 1, 0)

            process(t1, 1)

    return run


def kernel(x, y, x0, y0, image, pixelscale, scale):
    h, w = image.shape
    n = x.size
    xf = x.reshape(-1)
    yf = y.reshape(-1)
    img = image.reshape(-1)
    fov_x = pixelscale * w
    fov_y = pixelscale * h
    ax = jnp.float32(w) / fov_x
    ay = jnp.float32(h) / fov_y
    bx = jnp.float32((w - 1) * 0.5) - x0 * ax
    by = jnp.float32((h - 1) * 0.5) - y0 * ay
    params = jnp.concatenate(
        [jnp.full((L,), v, jnp.float32) for v in (ax, bx, ay, by, scale)])
    out = _build(n, h, w)(xf, yf, img, params)
    return out.reshape(x.shape)
